# initial kernel scaffold (unmeasured)
import jax
import jax.numpy as jnp
from jax import lax
from jax.experimental import pallas as pl
from jax.experimental.pallas import tpu as pltpu

N_DEV = 8
SQ = 2048
D_MODEL = 1024
H_LOC = 8
DH = 128
NG = 4
NB = SQ // 64
JPG = NB // NG
GROUP = JPG * 64
CHUNK = SQ // N_DEV
SCALE = 0.08838834764831843


def _body(x_ref, wq_ref, k_ref, v_ref, wo_ref, out_ref,
          xp, kp, vp, qp, ctxp, rsbuf,
          rs_send, rs_recv, ag_send, ag_recv):
    my = lax.axis_index("i")
    right = lax.rem(my + 1, N_DEV)

    for g in range(NG):
        for j in range(JPG):
            b = g + NG * j
            dst = slice(g * GROUP + j * 64, g * GROUP + (j + 1) * 64)
            src = slice(b * 64, (b + 1) * 64)
            xp[dst, :] = x_ref[src, :]
            kp[dst, :] = k_ref[src, :]
            vp[dst, :] = v_ref[src, :]

    qp[:, :] = jnp.dot(
        xp[:, :], wq_ref[:, :], preferred_element_type=jnp.float32
    ).astype(jnp.bfloat16)

    for g in range(NG):
        rows = slice(g * GROUP, (g + 1) * GROUP)
        for h in range(H_LOC):
            cols = slice(h * DH, (h + 1) * DH)
            s = lax.dot_general(
                qp[rows, cols], kp[rows, cols],
                (((1,), (1,)), ((), ())),
                preferred_element_type=jnp.float32,
            ) * SCALE
            m = jnp.max(s, axis=1, keepdims=True)
            e = jnp.exp(s - m)
            w = (e / jnp.sum(e, axis=1, keepdims=True)).astype(jnp.bfloat16)
            ctxp[rows, cols] = jnp.dot(
                w, vp[rows, cols], preferred_element_type=jnp.float32
            ).astype(jnp.bfloat16)

    for g in range(NG):
        p = jnp.dot(
            ctxp[g * GROUP:(g + 1) * GROUP, :], wo_ref[:, :],
            preferred_element_type=jnp.float32,
        )
        for j in range(JPG):
            b = g + NG * j
            out_ref[b * 64:(b + 1) * 64, :] = p[j * 64:(j + 1) * 64, :]

    for s in range(N_DEV - 1):
        cs = lax.rem(my - s + N_DEV, N_DEV)
        rdma = pltpu.make_async_remote_copy(
            src_ref=out_ref.at[pl.ds(cs * CHUNK, CHUNK), :],
            dst_ref=rsbuf.at[s],
            send_sem=rs_send.at[s],
            recv_sem=rs_recv.at[s],
            device_id=(right,),
            device_id_type=pl.DeviceIdType.MESH,
        )
        rdma.start()
        rdma.wait()
        cr = lax.rem(my - 1 - s + N_DEV, N_DEV)
        r = pl.ds(cr * CHUNK, CHUNK)
        out_ref[r, :] = out_ref[r, :] + rsbuf[s, :, :]


    for s in range(N_DEV - 1):
        cs = lax.rem(my + 1 - s + N_DEV, N_DEV)
        rdma = pltpu.make_async_remote_copy(
            src_ref=out_ref.at[pl.ds(cs * CHUNK, CHUNK), :],
            dst_ref=out_ref.at[pl.ds(cs * CHUNK, CHUNK), :],
            send_sem=ag_send.at[s],
            recv_sem=ag_recv.at[s],
            device_id=(right,),
            device_id_type=pl.DeviceIdType.MESH,
        )
        rdma.start()
        rdma.wait()


def kernel(x, Wq, K_ext, V_ext, Wo):
    idx = lax.axis_index("i")
    xb = x[0].astype(jnp.bfloat16)
    wqb = Wq.astype(jnp.bfloat16)
    kl = lax.dynamic_slice_in_dim(K_ext[0], idx * H_LOC, H_LOC, axis=1)
    vl = lax.dynamic_slice_in_dim(V_ext[0], idx * H_LOC, H_LOC, axis=1)
    kl = kl.reshape(SQ, H_LOC * DH).astype(jnp.bfloat16)
    vl = vl.reshape(SQ, H_LOC * DH).astype(jnp.bfloat16)
    wob = Wo.astype(jnp.bfloat16)

    out2d = pl.pallas_call(
        _body,
        out_shape=jax.ShapeDtypeStruct((SQ, D_MODEL), jnp.float32),
        in_specs=[pl.BlockSpec(memory_space=pltpu.VMEM)] * 5,
        out_specs=pl.BlockSpec(memory_space=pltpu.VMEM),
        scratch_shapes=[
            pltpu.VMEM((SQ, D_MODEL), jnp.bfloat16),
            pltpu.VMEM((SQ, H_LOC * DH), jnp.bfloat16),
            pltpu.VMEM((SQ, H_LOC * DH), jnp.bfloat16),
            pltpu.VMEM((SQ, D_MODEL), jnp.bfloat16),
            pltpu.VMEM((SQ, D_MODEL), jnp.bfloat16),
            pltpu.VMEM((N_DEV - 1, CHUNK, D_MODEL), jnp.float32),
            pltpu.SemaphoreType.DMA((N_DEV - 1,)),
            pltpu.SemaphoreType.DMA((N_DEV - 1,)),
            pltpu.SemaphoreType.DMA((N_DEV - 1,)),
            pltpu.SemaphoreType.DMA((N_DEV - 1,)),
        ],
        compiler_params=pltpu.CompilerParams(
            collective_id=0,
            vmem_limit_bytes=128 * 1024 * 1024,
        ),
    )(xb, wqb, kl, vl, wob)
    return out2d.reshape(1, SQ, D_MODEL)


# baseline (device time: 257850 ns/iter reference)
import jax
import jax.numpy as jnp
from jax import lax
from jax.experimental import pallas as pl
from jax.experimental.pallas import tpu as pltpu

N_DEV = 8
SQ = 2048
D_MODEL = 1024
H_LOC = 8
DH = 128
NG = 4
NB = SQ // 64
JPG = NB // NG
GROUP = JPG * 64
CHUNK = SQ // N_DEV
SCALE = 0.08838834764831843


def _body(x_ref, wq_ref, k_ref, v_ref, wo_ref, out_ref,
          xp, kp, vp, qp, ctxp, rsbuf,
          rs_send, rs_recv, ag_send, ag_recv):
    my = lax.axis_index("i")
    right = lax.rem(my + 1, N_DEV)

    for g in range(NG):
        for j in range(JPG):
            b = g + NG * j
            dst = slice(g * GROUP + j * 64, g * GROUP + (j + 1) * 64)
            src = slice(b * 64, (b + 1) * 64)
            xp[dst, :] = x_ref[src, :]
            kp[dst, :] = k_ref[src, :]
            vp[dst, :] = v_ref[src, :]

    qp[:, :] = jnp.dot(
        xp[:, :], wq_ref[:, :], preferred_element_type=jnp.float32
    ).astype(jnp.bfloat16)

    for g in range(NG):
        rows = slice(g * GROUP, (g + 1) * GROUP)
        for h in range(H_LOC):
            cols = slice(h * DH, (h + 1) * DH)
            s = lax.dot_general(
                qp[rows, cols], kp[rows, cols],
                (((1,), (1,)), ((), ())),
                preferred_element_type=jnp.float32,
            ) * SCALE
            m = jnp.max(s, axis=1, keepdims=True)
            e = jnp.exp(s - m)
            w = (e / jnp.sum(e, axis=1, keepdims=True)).astype(jnp.bfloat16)
            ctxp[rows, cols] = jnp.dot(
                w, vp[rows, cols], preferred_element_type=jnp.float32
            ).astype(jnp.bfloat16)

    for g in range(NG):
        p = jnp.dot(
            ctxp[g * GROUP:(g + 1) * GROUP, :], wo_ref[:, :],
            preferred_element_type=jnp.float32,
        )
        for j in range(JPG):
            b = g + NG * j
            out_ref[b * 64:(b + 1) * 64, :] = p[j * 64:(j + 1) * 64, :]

    for s in range(N_DEV - 1):
        cs = lax.rem(my - s + N_DEV, N_DEV)
        rdma = pltpu.make_async_remote_copy(
            src_ref=out_ref.at[pl.ds(cs * CHUNK, CHUNK), :],
            dst_ref=rsbuf.at[s],
            send_sem=rs_send.at[s],
            recv_sem=rs_recv.at[s],
            device_id=(right,),
            device_id_type=pl.DeviceIdType.MESH,
        )
        rdma.start()
        rdma.wait()
        cr = lax.rem(my - 1 - s + N_DEV, N_DEV)
        r = pl.ds(cr * CHUNK, CHUNK)
        out_ref[r, :] = out_ref[r, :] + rsbuf[s, :, :]


    for s in range(N_DEV - 1):
        cs = lax.rem(my + 1 - s + N_DEV, N_DEV)
        rdma = pltpu.make_async_remote_copy(
            src_ref=out_ref.at[pl.ds(cs * CHUNK, CHUNK), :],
            dst_ref=out_ref.at[pl.ds(cs * CHUNK, CHUNK), :],
            send_sem=ag_send.at[s],
            recv_sem=ag_recv.at[s],
            device_id=(right,),
            device_id_type=pl.DeviceIdType.MESH,
        )
        rdma.start()
        rdma.wait()


def kernel(x, Wq, K_ext, V_ext, Wo):
    idx = lax.axis_index("i")
    xb = x[0].astype(jnp.bfloat16)
    wqb = Wq.astype(jnp.bfloat16)
    kl = lax.dynamic_slice_in_dim(K_ext[0], idx * H_LOC, H_LOC, axis=1)
    vl = lax.dynamic_slice_in_dim(V_ext[0], idx * H_LOC, H_LOC, axis=1)
    kl = kl.reshape(SQ, H_LOC * DH).astype(jnp.bfloat16)
    vl = vl.reshape(SQ, H_LOC * DH).astype(jnp.bfloat16)
    wob = Wo.astype(jnp.bfloat16)

    out2d = pl.pallas_call(
        _body,
        out_shape=jax.ShapeDtypeStruct((SQ, D_MODEL), jnp.float32),
        in_specs=[pl.BlockSpec(memory_space=pltpu.VMEM)] * 5,
        out_specs=pl.BlockSpec(memory_space=pltpu.VMEM),
        scratch_shapes=[
            pltpu.VMEM((SQ, D_MODEL), jnp.bfloat16),
            pltpu.VMEM((SQ, H_LOC * DH), jnp.bfloat16),
            pltpu.VMEM((SQ, H_LOC * DH), jnp.bfloat16),
            pltpu.VMEM((SQ, D_MODEL), jnp.bfloat16),
            pltpu.VMEM((SQ, D_MODEL), jnp.bfloat16),
            pltpu.VMEM((N_DEV - 1, CHUNK, D_MODEL), jnp.float32),
            pltpu.SemaphoreType.DMA((N_DEV - 1,)),
            pltpu.SemaphoreType.DMA((N_DEV - 1,)),
            pltpu.SemaphoreType.DMA((N_DEV - 1,)),
            pltpu.SemaphoreType.DMA((N_DEV - 1,)),
        ],
        compiler_params=pltpu.CompilerParams(
            vmem_limit_bytes=128 * 1024 * 1024,
        ),
    )(xb, wqb, kl, vl, wob)
    return out2d.reshape(1, SQ, D_MODEL)


# device time: 140924 ns/iter; 1.8297x vs baseline; 1.8297x over previous
import jax
import jax.numpy as jnp
from jax import lax
from jax.experimental import pallas as pl
from jax.experimental.pallas import tpu as pltpu

N_DEV = 8
SQ = 2048
D_MODEL = 1024
H_LOC = 8
DH = 128
NG = 4
NB = SQ // 64
JPG = NB // NG
GROUP = JPG * 64
CHUNK = SQ // N_DEV
SCALE = 0.08838834764831843


HC = 1024 // N_DEV


def _body(x_ref, wq_ref, k_ref, v_ref, wo_ref, out_ref,
          xp, kp, vp, qp, ctxp, rsbuf_r, rsbuf_l,
          rsr_send, rsr_recv, rsl_send, rsl_recv,
          agr_send, agr_recv, agl_send, agl_recv):
    my = lax.axis_index("i")
    right = lax.rem(my + 1, N_DEV)
    left = lax.rem(my - 1 + N_DEV, N_DEV)

    for g in range(NG):
        for j in range(JPG):
            b = g + NG * j
            dst = slice(g * GROUP + j * 64, g * GROUP + (j + 1) * 64)
            src = slice(b * 64, (b + 1) * 64)
            xp[dst, :] = x_ref[src, :]
            kp[dst, :] = k_ref[src, :]
            vp[dst, :] = v_ref[src, :]

    qp[:, :] = jnp.dot(
        xp[:, :], wq_ref[:, :], preferred_element_type=jnp.float32
    ).astype(jnp.bfloat16)

    for g in range(NG):
        rows = slice(g * GROUP, (g + 1) * GROUP)
        for h in range(H_LOC):
            cols = slice(h * DH, (h + 1) * DH)
            s = lax.dot_general(
                qp[rows, cols], kp[rows, cols],
                (((1,), (1,)), ((), ())),
                preferred_element_type=jnp.float32,
            ) * SCALE
            m = jnp.max(s, axis=1, keepdims=True)
            e = jnp.exp(s - m)
            w = (e / jnp.sum(e, axis=1, keepdims=True)).astype(jnp.bfloat16)
            ctxp[rows, cols] = jnp.dot(
                w, vp[rows, cols], preferred_element_type=jnp.float32
            ).astype(jnp.bfloat16)

    for g in range(NG):
        p = jnp.dot(
            ctxp[g * GROUP:(g + 1) * GROUP, :], wo_ref[:, :],
            preferred_element_type=jnp.float32,
        )
        for j in range(JPG):
            b = g + NG * j
            out_ref[b * 64:(b + 1) * 64, :] = p[j * 64:(j + 1) * 64, :].astype(
                jnp.bfloat16
            )


    for s in range(N_DEV - 1):
        csr = lax.rem(my - s + N_DEV, N_DEV)
        csl = lax.rem(my + s, N_DEV)
        rdma_r = pltpu.make_async_remote_copy(
            src_ref=out_ref.at[pl.ds(csr * HC, HC), :],
            dst_ref=rsbuf_r.at[s],
            send_sem=rsr_send.at[s],
            recv_sem=rsr_recv.at[s],
            device_id=(right,),
            device_id_type=pl.DeviceIdType.MESH,
        )
        rdma_l = pltpu.make_async_remote_copy(
            src_ref=out_ref.at[pl.ds(1024 + csl * HC, HC), :],
            dst_ref=rsbuf_l.at[s],
            send_sem=rsl_send.at[s],
            recv_sem=rsl_recv.at[s],
            device_id=(left,),
            device_id_type=pl.DeviceIdType.MESH,
        )
        rdma_r.start()
        rdma_l.start()
        rdma_r.wait()
        crr = lax.rem(my - 1 - s + N_DEV, N_DEV)
        rr = pl.ds(crr * HC, HC)
        out_ref[rr, :] = (
            out_ref[rr, :].astype(jnp.float32)
            + rsbuf_r[s, :, :].astype(jnp.float32)
        ).astype(jnp.bfloat16)
        rdma_l.wait()
        crl = lax.rem(my + 1 + s, N_DEV)
        rl = pl.ds(1024 + crl * HC, HC)
        out_ref[rl, :] = (
            out_ref[rl, :].astype(jnp.float32)
            + rsbuf_l[s, :, :].astype(jnp.float32)
        ).astype(jnp.bfloat16)


    for s in range(N_DEV - 1):
        csr = lax.rem(my + 1 - s + N_DEV, N_DEV)
        csl = lax.rem(my - 1 + s + N_DEV, N_DEV)
        rdma_r = pltpu.make_async_remote_copy(
            src_ref=out_ref.at[pl.ds(csr * HC, HC), :],
            dst_ref=out_ref.at[pl.ds(csr * HC, HC), :],
            send_sem=agr_send.at[s],
            recv_sem=agr_recv.at[s],
            device_id=(right,),
            device_id_type=pl.DeviceIdType.MESH,
        )
        rdma_l = pltpu.make_async_remote_copy(
            src_ref=out_ref.at[pl.ds(1024 + csl * HC, HC), :],
            dst_ref=out_ref.at[pl.ds(1024 + csl * HC, HC), :],
            send_sem=agl_send.at[s],
            recv_sem=agl_recv.at[s],
            device_id=(left,),
            device_id_type=pl.DeviceIdType.MESH,
        )
        rdma_r.start()
        rdma_l.start()
        rdma_r.wait()
        rdma_l.wait()


def kernel(x, Wq, K_ext, V_ext, Wo):
    idx = lax.axis_index("i")
    xb = x[0].astype(jnp.bfloat16)
    wqb = Wq.astype(jnp.bfloat16)
    kl = lax.dynamic_slice_in_dim(K_ext[0], idx * H_LOC, H_LOC, axis=1)
    vl = lax.dynamic_slice_in_dim(V_ext[0], idx * H_LOC, H_LOC, axis=1)
    kl = kl.reshape(SQ, H_LOC * DH).astype(jnp.bfloat16)
    vl = vl.reshape(SQ, H_LOC * DH).astype(jnp.bfloat16)
    wob = Wo.astype(jnp.bfloat16)

    out2d = pl.pallas_call(
        _body,
        out_shape=jax.ShapeDtypeStruct((SQ, D_MODEL), jnp.bfloat16),
        in_specs=[pl.BlockSpec(memory_space=pltpu.VMEM)] * 5,
        out_specs=pl.BlockSpec(memory_space=pltpu.VMEM),
        scratch_shapes=[
            pltpu.VMEM((SQ, D_MODEL), jnp.bfloat16),
            pltpu.VMEM((SQ, H_LOC * DH), jnp.bfloat16),
            pltpu.VMEM((SQ, H_LOC * DH), jnp.bfloat16),
            pltpu.VMEM((SQ, D_MODEL), jnp.bfloat16),
            pltpu.VMEM((SQ, D_MODEL), jnp.bfloat16),
            pltpu.VMEM((N_DEV - 1, HC, D_MODEL), jnp.bfloat16),
            pltpu.VMEM((N_DEV - 1, HC, D_MODEL), jnp.bfloat16),
            pltpu.SemaphoreType.DMA((N_DEV - 1,)),
            pltpu.SemaphoreType.DMA((N_DEV - 1,)),
            pltpu.SemaphoreType.DMA((N_DEV - 1,)),
            pltpu.SemaphoreType.DMA((N_DEV - 1,)),
            pltpu.SemaphoreType.DMA((N_DEV - 1,)),
            pltpu.SemaphoreType.DMA((N_DEV - 1,)),
            pltpu.SemaphoreType.DMA((N_DEV - 1,)),
            pltpu.SemaphoreType.DMA((N_DEV - 1,)),
        ],
        compiler_params=pltpu.CompilerParams(
            vmem_limit_bytes=128 * 1024 * 1024,
        ),
    )(xb, wqb, kl, vl, wob)
    return out2d.reshape(1, SQ, D_MODEL)


# device time: 112483 ns/iter; 2.2923x vs baseline; 1.2528x over previous
import jax
import jax.numpy as jnp
from jax import lax
from jax.experimental import pallas as pl
from jax.experimental.pallas import tpu as pltpu

N_DEV = 8
SQ = 2048
D_MODEL = 1024
H_LOC = 8
DH = 128
NG = 4
NB = SQ // 64
JPG = NB // NG
GROUP = JPG * 64
CHUNK = SQ // N_DEV
SCALE = 0.08838834764831843


HD_INST = (
    (0, 768, (1, 3, 4)),
    (768, 768, (3, 4, 1)),
    (1536, 512, (4, 1, 3)),
)


def _tbit(my, m):
    p0 = lax.rem(my, 2)
    p1 = lax.rem(my // 2, 2)
    p2 = my // 4
    if m == 1:
        return lax.rem(p0 + p1, 2)
    if m == 3:
        return p1
    return p2


def _body(x_ref, wq_ref, k_ref, v_ref, wo_ref, out_ref,
          xp, kp, vp, qp, ctxp, *comm):
    bufs = [[comm[i * 3 + r] for r in range(3)] for i in range(3)]
    rs_send, rs_recv, ag_send, ag_recv = comm[9:13]
    my = lax.axis_index("i")

    for g in range(NG):
        for j in range(JPG):
            b = g + NG * j
            dst = slice(g * GROUP + j * 64, g * GROUP + (j + 1) * 64)
            src = slice(b * 64, (b + 1) * 64)
            xp[dst, :] = x_ref[src, :]
            kp[dst, :] = k_ref[src, :]
            vp[dst, :] = v_ref[src, :]

    qp[:, :] = jnp.dot(
        xp[:, :], wq_ref[:, :], preferred_element_type=jnp.float32
    ).astype(jnp.bfloat16)

    for g in range(NG):
        rows = slice(g * GROUP, (g + 1) * GROUP)
        for h in range(H_LOC):
            cols = slice(h * DH, (h + 1) * DH)
            s = lax.dot_general(
                qp[rows, cols], kp[rows, cols],
                (((1,), (1,)), ((), ())),
                preferred_element_type=jnp.float32,
            ) * SCALE
            m = jnp.max(s, axis=1, keepdims=True)
            e = jnp.exp(s - m)
            w = (e / jnp.sum(e, axis=1, keepdims=True)).astype(jnp.bfloat16)
            ctxp[rows, cols] = jnp.dot(
                w, vp[rows, cols], preferred_element_type=jnp.float32
            ).astype(jnp.bfloat16)

    for g in range(NG):
        p = jnp.dot(
            ctxp[g * GROUP:(g + 1) * GROUP, :], wo_ref[:, :],
            preferred_element_type=jnp.float32,
        )
        for j in range(JPG):
            b = g + NG * j
            out_ref[b * 64:(b + 1) * 64, :] = p[j * 64:(j + 1) * 64, :].astype(
                jnp.bfloat16
            )

    offs = [base for base, _, _ in HD_INST]
    for r in range(3):
        rdmas = []
        for i, (base, size, masks) in enumerate(HD_INST):
            m = masks[r]
            half = size >> (r + 1)
            t = _tbit(my, m)
            partner = jnp.bitwise_xor(my, m)
            send_off = offs[i] + (1 - t) * half
            rdma = pltpu.make_async_remote_copy(
                src_ref=out_ref.at[pl.ds(send_off, half), :],
                dst_ref=bufs[i][r],
                send_sem=rs_send.at[i, r],
                recv_sem=rs_recv.at[i, r],
                device_id=(partner,),
                device_id_type=pl.DeviceIdType.MESH,
            )
            rdma.start()
            rdmas.append((rdma, i, half, t))
        for rdma, i, half, t in rdmas:
            rdma.wait()
            keep_off = offs[i] + t * half
            reg = pl.ds(keep_off, half)
            out_ref[reg, :] = (
                out_ref[reg, :].astype(jnp.float32)
                + bufs[i][r][:, :].astype(jnp.float32)
            ).astype(jnp.bfloat16)
            offs[i] = keep_off


    for r in (2, 1, 0):
        rdmas = []
        for i, (base, size, masks) in enumerate(HD_INST):
            m = masks[r]
            rsz = size >> (r + 1)
            t = _tbit(my, m)
            partner = jnp.bitwise_xor(my, m)
            rdma = pltpu.make_async_remote_copy(
                src_ref=out_ref.at[pl.ds(offs[i], rsz), :],
                dst_ref=out_ref.at[pl.ds(offs[i], rsz), :],
                send_sem=ag_send.at[i, r],
                recv_sem=ag_recv.at[i, r],
                device_id=(partner,),
                device_id_type=pl.DeviceIdType.MESH,
            )
            rdma.start()
            rdmas.append((rdma, i, rsz, t))
        for rdma, i, rsz, t in rdmas:
            rdma.wait()
            offs[i] = offs[i] - t * rsz


def kernel(x, Wq, K_ext, V_ext, Wo):
    idx = lax.axis_index("i")
    xb = x[0].astype(jnp.bfloat16)
    wqb = Wq.astype(jnp.bfloat16)
    kl = lax.dynamic_slice_in_dim(K_ext[0], idx * H_LOC, H_LOC, axis=1)
    vl = lax.dynamic_slice_in_dim(V_ext[0], idx * H_LOC, H_LOC, axis=1)
    kl = kl.reshape(SQ, H_LOC * DH).astype(jnp.bfloat16)
    vl = vl.reshape(SQ, H_LOC * DH).astype(jnp.bfloat16)
    wob = Wo.astype(jnp.bfloat16)

    out2d = pl.pallas_call(
        _body,
        out_shape=jax.ShapeDtypeStruct((SQ, D_MODEL), jnp.bfloat16),
        in_specs=[pl.BlockSpec(memory_space=pltpu.VMEM)] * 5,
        out_specs=pl.BlockSpec(memory_space=pltpu.VMEM),
        scratch_shapes=[
            pltpu.VMEM((SQ, D_MODEL), jnp.bfloat16),
            pltpu.VMEM((SQ, H_LOC * DH), jnp.bfloat16),
            pltpu.VMEM((SQ, H_LOC * DH), jnp.bfloat16),
            pltpu.VMEM((SQ, D_MODEL), jnp.bfloat16),
            pltpu.VMEM((SQ, D_MODEL), jnp.bfloat16),
        ] + [
            pltpu.VMEM((size >> (r + 1), D_MODEL), jnp.bfloat16)
            for _, size, _ in HD_INST
            for r in range(3)
        ] + [
            pltpu.SemaphoreType.DMA((3, 3)),
            pltpu.SemaphoreType.DMA((3, 3)),
            pltpu.SemaphoreType.DMA((3, 3)),
            pltpu.SemaphoreType.DMA((3, 3)),
        ],
        compiler_params=pltpu.CompilerParams(
            vmem_limit_bytes=128 * 1024 * 1024,
        ),
    )(xb, wqb, kl, vl, wob)
    return out2d.reshape(1, SQ, D_MODEL)


# device time: 103050 ns/iter; 2.5022x vs baseline; 1.0915x over previous
import jax
import jax.numpy as jnp
from jax import lax
from jax.experimental import pallas as pl
from jax.experimental.pallas import tpu as pltpu

N_DEV = 8
SQ = 2048
D_MODEL = 1024
H_LOC = 8
DH = 128
NG = 4
NB = SQ // 64
JPG = NB // NG
GROUP = JPG * 64
CHUNK = SQ // N_DEV
SCALE = 0.08838834764831843


HD_INST = (
    (0, 768, (1, 3, 4)),
    (768, 768, (3, 4, 1)),
    (1536, 512, (4, 1, 3)),
)


def _tbit(my, m):
    p0 = lax.rem(my, 2)
    p1 = lax.rem(my // 2, 2)
    p2 = my // 4
    if m == 1:
        return lax.rem(p0 + p1, 2)
    if m == 3:
        return p1
    return p2


def _body(x_ref, wq_ref, k_ref, v_ref, wo_ref, out_ref,
          xp, kp, vp, qp, ctxp, kbuf, vbuf, *comm):
    bufs = [[comm[i * 3 + r] for r in range(3)] for i in range(3)]
    rs_send, rs_recv, ag_send, ag_recv = comm[9:13]
    k_sem, v_sem = comm[13], comm[14]
    my = lax.axis_index("i")

    k_dma = pltpu.make_async_copy(
        k_ref.at[0, :, pl.ds(my * H_LOC, H_LOC), :], kbuf, k_sem
    )
    v_dma = pltpu.make_async_copy(
        v_ref.at[0, :, pl.ds(my * H_LOC, H_LOC), :], vbuf, v_sem
    )
    k_dma.start()
    v_dma.start()

    for g in range(NG):
        for j in range(JPG):
            b = g + NG * j
            dst = slice(g * GROUP + j * 64, g * GROUP + (j + 1) * 64)
            src = slice(b * 64, (b + 1) * 64)
            xp[dst, :] = x_ref[0, src.start:src.stop, :].astype(jnp.bfloat16)

    qp[:, :] = jnp.dot(
        xp[:, :], wq_ref[:, :].astype(jnp.bfloat16),
        preferred_element_type=jnp.float32,
    ).astype(jnp.bfloat16)

    k_dma.wait()
    v_dma.wait()
    for g in range(NG):
        for j in range(JPG):
            b = g + NG * j
            dst = slice(g * GROUP + j * 64, g * GROUP + (j + 1) * 64)
            src = slice(b * 64, (b + 1) * 64)
            kp[dst, :, :] = kbuf[src, :, :].astype(jnp.bfloat16)
            vp[dst, :, :] = vbuf[src, :, :].astype(jnp.bfloat16)

    for g in range(NG):
        rows = slice(g * GROUP, (g + 1) * GROUP)
        for h in range(H_LOC):
            cols = slice(h * DH, (h + 1) * DH)
            s = lax.dot_general(
                qp[rows, cols], kp[rows, h, :],
                (((1,), (1,)), ((), ())),
                preferred_element_type=jnp.float32,
            ) * SCALE
            m = jnp.max(s, axis=1, keepdims=True)
            e = jnp.exp(s - m)
            w = (e / jnp.sum(e, axis=1, keepdims=True)).astype(jnp.bfloat16)
            ctxp[rows, cols] = jnp.dot(
                w, vp[rows, h, :], preferred_element_type=jnp.float32
            ).astype(jnp.bfloat16)

    wo_b = wo_ref[:, :].astype(jnp.bfloat16)
    for g in range(NG):
        p = jnp.dot(
            ctxp[g * GROUP:(g + 1) * GROUP, :], wo_b,
            preferred_element_type=jnp.float32,
        )
        for j in range(JPG):
            b = g + NG * j
            out_ref[b * 64:(b + 1) * 64, :] = p[j * 64:(j + 1) * 64, :].astype(
                jnp.bfloat16
            )

    offs = [base for base, _, _ in HD_INST]
    for r in range(3):
        rdmas = []
        for i, (base, size, masks) in enumerate(HD_INST):
            m = masks[r]
            half = size >> (r + 1)
            t = _tbit(my, m)
            partner = jnp.bitwise_xor(my, m)
            send_off = offs[i] + (1 - t) * half
            rdma = pltpu.make_async_remote_copy(
                src_ref=out_ref.at[pl.ds(send_off, half), :],
                dst_ref=bufs[i][r],
                send_sem=rs_send.at[i, r],
                recv_sem=rs_recv.at[i, r],
                device_id=(partner,),
                device_id_type=pl.DeviceIdType.MESH,
            )
            rdma.start()
            rdmas.append((rdma, i, half, t))
        for rdma, i, half, t in rdmas:
            rdma.wait()
            keep_off = offs[i] + t * half
            reg = pl.ds(keep_off, half)
            out_ref[reg, :] = (
                out_ref[reg, :].astype(jnp.float32)
                + bufs[i][r][:, :].astype(jnp.float32)
            ).astype(jnp.bfloat16)
            offs[i] = keep_off


    for r in (2, 1, 0):
        rdmas = []
        for i, (base, size, masks) in enumerate(HD_INST):
            m = masks[r]
            rsz = size >> (r + 1)
            t = _tbit(my, m)
            partner = jnp.bitwise_xor(my, m)
            rdma = pltpu.make_async_remote_copy(
                src_ref=out_ref.at[pl.ds(offs[i], rsz), :],
                dst_ref=out_ref.at[pl.ds(offs[i], rsz), :],
                send_sem=ag_send.at[i, r],
                recv_sem=ag_recv.at[i, r],
                device_id=(partner,),
                device_id_type=pl.DeviceIdType.MESH,
            )
            rdma.start()
            rdmas.append((rdma, i, rsz, t))
        for rdma, i, rsz, t in rdmas:
            rdma.wait()
            offs[i] = offs[i] - t * rsz


def kernel(x, Wq, K_ext, V_ext, Wo):
    out2d = pl.pallas_call(
        _body,
        out_shape=jax.ShapeDtypeStruct((SQ, D_MODEL), jnp.bfloat16),
        in_specs=[
            pl.BlockSpec(memory_space=pltpu.VMEM),
            pl.BlockSpec(memory_space=pltpu.VMEM),
            pl.BlockSpec(memory_space=pltpu.MemorySpace.HBM),
            pl.BlockSpec(memory_space=pltpu.MemorySpace.HBM),
            pl.BlockSpec(memory_space=pltpu.VMEM),
        ],
        out_specs=pl.BlockSpec(memory_space=pltpu.VMEM),
        scratch_shapes=[
            pltpu.VMEM((SQ, D_MODEL), jnp.bfloat16),
            pltpu.VMEM((SQ, H_LOC, DH), jnp.bfloat16),
            pltpu.VMEM((SQ, H_LOC, DH), jnp.bfloat16),
            pltpu.VMEM((SQ, D_MODEL), jnp.bfloat16),
            pltpu.VMEM((SQ, D_MODEL), jnp.bfloat16),
            pltpu.VMEM((SQ, H_LOC, DH), jnp.float32),
            pltpu.VMEM((SQ, H_LOC, DH), jnp.float32),
        ] + [
            pltpu.VMEM((size >> (r + 1), D_MODEL), jnp.bfloat16)
            for _, size, _ in HD_INST
            for r in range(3)
        ] + [
            pltpu.SemaphoreType.DMA((3, 3)),
            pltpu.SemaphoreType.DMA((3, 3)),
            pltpu.SemaphoreType.DMA((3, 3)),
            pltpu.SemaphoreType.DMA((3, 3)),
            pltpu.SemaphoreType.DMA,
            pltpu.SemaphoreType.DMA,
        ],
        compiler_params=pltpu.CompilerParams(
            vmem_limit_bytes=128 * 1024 * 1024,
        ),
    )(x, Wq, K_ext, V_ext, Wo)
    return out2d.reshape(1, SQ, D_MODEL)
